# Initial kernel scaffold; baseline (speedup 1.0000x reference)
import jax
import jax.numpy as jnp
from jax.experimental import pallas as pl


def kernel(pos_u, pos_w, neg_u, neg_w, n, u_table, w_table):
    raise NotImplementedError("write your pallas kernel here")


# speedup vs baseline: 1.7592x; 1.7592x over previous
import functools

import jax
import jax.numpy as jnp
from jax import lax
from jax.experimental import pallas as pl
from jax.experimental.pallas import tpu as pltpu
from jax.experimental.pallas import tpu_sc as plsc

_B = 16384
_CTX = 20
_D = 64
_TOT = 2 * _B
_NC, _NS = 2, 16
_NW = _NC * _NS
_PER_W = _TOT // _NW
_G = 128
_E = 32
_GPC = _E * _CTX // _G
_CHUNKS = _PER_W // _E
_WCH = _PER_W // _G
_DW = _D // 16
_LANES = 16


def _sc_gather_sum(u_idx, w_idx, u_table, w_table):
    mesh = plsc.VectorSubcoreMesh(core_axis_name="c", subcore_axis_name="s")

    @functools.partial(
        pl.kernel,
        compiler_params=pltpu.CompilerParams(use_tc_tiling_on_sc=False),
        out_type=(
            jax.ShapeDtypeStruct((_TOT, _D), jnp.float32),
            jax.ShapeDtypeStruct((_TOT, _D), jnp.float32),
        ),
        mesh=mesh,
        scratch_types=[
            pltpu.VMEM((_CHUNKS * _GPC, _G), jnp.int32),
            pltpu.VMEM((_WCH, _G), jnp.int32),
            pltpu.VMEM((_E * _CTX, _D), jnp.float32),
            pltpu.VMEM((_G, _D), jnp.float32),
            pltpu.VMEM((_E, _D), jnp.float32),
            pltpu.SemaphoreType.DMA,
        ],
    )
    def k(uidx_hbm, widx_hbm, utab_hbm, wtab_hbm, uemb_hbm, wemb_hbm,
          uidx_v, widx_v, rows_v, wrows_v, out_v, sem):
        wid = lax.axis_index("s") * _NC + lax.axis_index("c")
        base = wid * _PER_W
        pltpu.sync_copy(uidx_hbm.at[wid], uidx_v)
        pltpu.sync_copy(widx_hbm.at[wid], widx_v)

        @pl.loop(0, _CHUNKS)
        def _chunk(ck):
            cps = [
                pltpu.async_copy(
                    utab_hbm.at[uidx_v.at[ck * _GPC + j]],
                    rows_v.at[pl.ds(j * _G, _G)],
                    sem,
                )
                for j in range(_GPC)
            ]
            for cp in cps:
                cp.wait()

            @pl.loop(0, _E)
            def _ex(e):
                r0 = e * _CTX
                for d in range(_DW):
                    sl = pl.ds(d * _LANES, _LANES)
                    acc = rows_v[r0, sl]
                    for c in range(1, _CTX):
                        acc = acc + rows_v[r0 + c, sl]
                    out_v[e, sl] = acc

            pltpu.sync_copy(out_v, uemb_hbm.at[pl.ds(base + ck * _E, _E)])

        @pl.loop(0, _WCH)
        def _wch(j):
            pltpu.async_copy(wtab_hbm.at[widx_v.at[j]], wrows_v, sem).wait()
            pltpu.sync_copy(wrows_v, wemb_hbm.at[pl.ds(base + j * _G, _G)])

    return k(u_idx, w_idx, u_table, w_table)


def _tc_loss(u_emb, w_emb):
    def body(u_ref, w_ref, o_ref):
        s = jnp.sum(u_ref[...] * w_ref[...], axis=1, keepdims=True)
        row = lax.broadcasted_iota(jnp.int32, (_TOT, 1), 0)
        z = jnp.where(row < _B, -s, s)
        o_ref[...] = jnp.sum(jax.nn.log_sigmoid(z)).reshape(1, 1)

    return pl.pallas_call(
        body,
        out_shape=jax.ShapeDtypeStruct((1, 1), jnp.float32),
    )(u_emb, w_emb)


def kernel(pos_u, pos_w, neg_u, neg_w, n, u_table, w_table):
    u_idx = jnp.concatenate(
        [pos_u.reshape(-1), neg_u.reshape(-1)]
    ).astype(jnp.int32).reshape(_NW, _CHUNKS * _GPC, _G)
    w_idx = jnp.concatenate([pos_w, neg_w]).astype(jnp.int32).reshape(
        _NW, _WCH, _G)
    u_emb, w_emb = _sc_gather_sum(u_idx, w_idx, u_table, w_table)
    loss = _tc_loss(u_emb, w_emb)[0, 0]
    return -1.0 * loss / n


# speedup vs baseline: 2.2218x; 1.2629x over previous
import functools

import jax
import jax.numpy as jnp
from jax import lax
from jax.experimental import pallas as pl
from jax.experimental.pallas import tpu as pltpu
from jax.experimental.pallas import tpu_sc as plsc

_B = 16384
_CTX = 20
_D = 64
_TOT = 2 * _B
_NC, _NS = 2, 16
_NW = _NC * _NS
_PER_W = _TOT // _NW
_G = 128
_E = 32
_GPC = _E * _CTX // _G
_CHUNKS = _PER_W // _E
_DW = _D // 16
_LANES = 16
_ROWS = 199999


def _sc_gather_sum(u_idx, w_idx, tab2):
    mesh = plsc.VectorSubcoreMesh(core_axis_name="c", subcore_axis_name="s")

    @functools.partial(
        pl.kernel,
        compiler_params=pltpu.CompilerParams(use_tc_tiling_on_sc=False),
        out_type=jax.ShapeDtypeStruct((_TOT, 2 * _D), jnp.float32),
        mesh=mesh,
        scratch_types=[
            pltpu.VMEM((_CHUNKS * _GPC, _G), jnp.int32),
            pltpu.VMEM((_CHUNKS, _E), jnp.int32),
            pltpu.VMEM((_E * _CTX, _D), jnp.float32),
            pltpu.VMEM((_E, _D), jnp.float32),
            pltpu.VMEM((_E, 2 * _D), jnp.float32),
            pltpu.SemaphoreType.DMA,
        ],
    )
    def k(uidx_hbm, widx_hbm, tab_hbm, out_hbm,
          uidx_v, widx_v, rows_v, wrows_v, out_v, sem):
        wid = lax.axis_index("s") * _NC + lax.axis_index("c")
        base = wid * _PER_W
        pltpu.sync_copy(uidx_hbm.at[wid], uidx_v)
        pltpu.sync_copy(widx_hbm.at[wid], widx_v)

        @pl.loop(0, _CHUNKS)
        def _chunk(ck):
            cps = [
                pltpu.async_copy(
                    tab_hbm.at[uidx_v.at[ck * _GPC + j]],
                    rows_v.at[pl.ds(j * _G, _G)],
                    sem,
                )
                for j in range(_GPC)
            ]
            cps.append(pltpu.async_copy(tab_hbm.at[widx_v.at[ck]], wrows_v, sem))
            for cp in cps:
                cp.wait()

            @pl.loop(0, _E)
            def _ex(e):
                r0 = e * _CTX
                for d in range(_DW):
                    sl = pl.ds(d * _LANES, _LANES)
                    acc = rows_v[r0, sl]
                    for c in range(1, _CTX):
                        acc = acc + rows_v[r0 + c, sl]
                    out_v[e, sl] = acc
                    out_v[e, pl.ds(_D + d * _LANES, _LANES)] = wrows_v[e, sl]

            pltpu.sync_copy(out_v, out_hbm.at[pl.ds(base + ck * _E, _E)])

    return k(u_idx, w_idx, tab2)


def _tc_loss(uw_emb):
    def body(x_ref, o_ref):
        u = x_ref[:, : _D]
        w = x_ref[:, _D:]
        s = jnp.sum(u * w, axis=1, keepdims=True)
        row = lax.broadcasted_iota(jnp.int32, (_TOT, 1), 0)
        z = jnp.where(row < _B, -s, s)
        o_ref[...] = jnp.sum(jax.nn.log_sigmoid(z)).reshape(1, 1)

    return pl.pallas_call(
        body,
        out_shape=jax.ShapeDtypeStruct((1, 1), jnp.float32),
    )(uw_emb)


def kernel(pos_u, pos_w, neg_u, neg_w, n, u_table, w_table):
    u_idx = (2 * jnp.concatenate(
        [pos_u.reshape(-1), neg_u.reshape(-1)]
    ).astype(jnp.int32)).reshape(_NW, _CHUNKS * _GPC, _G)
    w_idx = (2 * jnp.concatenate([pos_w, neg_w]).astype(jnp.int32)
             + 1).reshape(_NW, _CHUNKS, _E)
    comb = jnp.concatenate(
        [jnp.pad(u_table, ((0, 1), (0, 0))), jnp.pad(w_table, ((0, 1), (0, 0)))],
        axis=1,
    )
    tab2 = comb.reshape(2 * (_ROWS + 1), _D)
    uw_emb = _sc_gather_sum(u_idx, w_idx, tab2)
    loss = _tc_loss(uw_emb)[0, 0]
    return -1.0 * loss / n


# speedup vs baseline: 2.7449x; 1.2355x over previous
import functools

import jax
import jax.numpy as jnp
from jax import lax
from jax.experimental import pallas as pl
from jax.experimental.pallas import tpu as pltpu
from jax.experimental.pallas import tpu_sc as plsc

_B = 16384
_CTX = 20
_D = 64
_TOT = 2 * _B
_NC, _NS = 2, 16
_NW = _NC * _NS
_PER_W = _TOT // _NW
_G = 128
_E = 32
_GPC = _E * _CTX // _G
_CHUNKS = _PER_W // _E
_DW = _D // 16
_LANES = 16
_ROWS = 199999


def _sc_gather_sum(u_idx, w_idx, tab2):
    mesh = plsc.VectorSubcoreMesh(core_axis_name="c", subcore_axis_name="s")

    @functools.partial(
        pl.kernel,
        compiler_params=pltpu.CompilerParams(use_tc_tiling_on_sc=False),
        out_type=jax.ShapeDtypeStruct((_TOT, 2 * _D), jnp.float32),
        mesh=mesh,
        scratch_types=[
            pltpu.VMEM((_CHUNKS * _GPC, _G), jnp.int32),
            pltpu.VMEM((_CHUNKS, _E), jnp.int32),
            pltpu.VMEM((_E * _CTX, _D), jnp.float32),
            pltpu.VMEM((_E * _CTX, _D), jnp.float32),
            pltpu.VMEM((_E, _D), jnp.float32),
            pltpu.VMEM((_E, _D), jnp.float32),
            pltpu.VMEM((_E, 2 * _D), jnp.float32),
            pltpu.VMEM((_E, 2 * _D), jnp.float32),
            pltpu.SemaphoreType.DMA,
            pltpu.SemaphoreType.DMA,
            pltpu.SemaphoreType.DMA,
            pltpu.SemaphoreType.DMA,
        ],
    )
    def k(uidx_hbm, widx_hbm, tab_hbm, out_hbm,
          uidx_v, widx_v, rows0, rows1, wrows0, wrows1, out0, out1,
          semg0, semg1, semo0, semo1):
        wid = lax.axis_index("s") * _NC + lax.axis_index("c")
        base = wid * _PER_W
        pltpu.sync_copy(uidx_hbm.at[wid], uidx_v)
        pltpu.sync_copy(widx_hbm.at[wid], widx_v)

        def issue(ck, rows_v, wrows_v, semg):
            for j in range(_GPC):
                pltpu.async_copy(
                    tab_hbm.at[uidx_v.at[ck * _GPC + j]],
                    rows_v.at[pl.ds(j * _G, _G)],
                    semg,
                )
            pltpu.async_copy(tab_hbm.at[widx_v.at[ck]], wrows_v, semg)

        def drain(rows_v, wrows_v, semg):
            pltpu.make_async_copy(
                tab_hbm.at[pl.ds(0, _E * _CTX)], rows_v, semg).wait()
            pltpu.make_async_copy(tab_hbm.at[pl.ds(0, _E)], wrows_v, semg).wait()

        def compute(rows_v, wrows_v, out_v):
            @pl.loop(0, _E)
            def _ex(e):
                r0 = e * _CTX
                for d in range(_DW):
                    sl = pl.ds(d * _LANES, _LANES)
                    acc = rows_v[r0, sl]
                    for c in range(1, _CTX):
                        acc = acc + rows_v[r0 + c, sl]
                    out_v[e, sl] = acc
                    out_v[e, pl.ds(_D + d * _LANES, _LANES)] = wrows_v[e, sl]

        def out_wait(out_v, semo):
            pltpu.make_async_copy(out_v, out_hbm.at[pl.ds(0, _E)], semo).wait()

        _H = _CHUNKS // 2
        issue(0, rows0, wrows0, semg0)

        @pl.loop(0, _H)
        def _pipe(kk):
            ck0 = 2 * kk
            issue(ck0 + 1, rows1, wrows1, semg1)
            drain(rows0, wrows0, semg0)

            @pl.when(kk > 0)
            def _():
                out_wait(out0, semo0)

            compute(rows0, wrows0, out0)
            pltpu.async_copy(out0, out_hbm.at[pl.ds(base + ck0 * _E, _E)], semo0)

            @pl.when(kk < _H - 1)
            def _():
                issue(ck0 + 2, rows0, wrows0, semg0)

            drain(rows1, wrows1, semg1)

            @pl.when(kk > 0)
            def _():
                out_wait(out1, semo1)

            compute(rows1, wrows1, out1)
            pltpu.async_copy(
                out1, out_hbm.at[pl.ds(base + (ck0 + 1) * _E, _E)], semo1)

        out_wait(out0, semo0)
        out_wait(out1, semo1)

    return k(u_idx, w_idx, tab2)


def _tc_loss(uw_emb):
    def body(x_ref, o_ref):
        u = x_ref[:, : _D]
        w = x_ref[:, _D:]
        s = jnp.sum(u * w, axis=1, keepdims=True)
        row = lax.broadcasted_iota(jnp.int32, (_TOT, 1), 0)
        z = jnp.where(row < _B, -s, s)
        o_ref[...] = jnp.sum(jax.nn.log_sigmoid(z)).reshape(1, 1)

    return pl.pallas_call(
        body,
        out_shape=jax.ShapeDtypeStruct((1, 1), jnp.float32),
    )(uw_emb)


def kernel(pos_u, pos_w, neg_u, neg_w, n, u_table, w_table):
    u_idx = (2 * jnp.concatenate(
        [pos_u.reshape(-1), neg_u.reshape(-1)]
    ).astype(jnp.int32)).reshape(_NW, _CHUNKS * _GPC, _G)
    w_idx = (2 * jnp.concatenate([pos_w, neg_w]).astype(jnp.int32)
             + 1).reshape(_NW, _CHUNKS, _E)
    comb = jnp.concatenate(
        [jnp.pad(u_table, ((0, 1), (0, 0))), jnp.pad(w_table, ((0, 1), (0, 0)))],
        axis=1,
    )
    tab2 = comb.reshape(2 * (_ROWS + 1), _D)
    uw_emb = _sc_gather_sum(u_idx, w_idx, tab2)
    loss = _tc_loss(uw_emb)[0, 0]
    return -1.0 * loss / n
